# Initial kernel scaffold; baseline (speedup 1.0000x reference)
import functools
import jax
import jax.numpy as jnp
from jax import lax
from jax.experimental import pallas as pl
from jax.experimental.pallas import tpu as pltpu
from jax.experimental.pallas import tpu_sc as plsc

B, H, W, EMB = 1024, 25, 25, 32
HW = H * W
NC, NS, L = 2, 16, 16
NW = NC * NS
B_PER_W = B // NW
GRP = 8
NGRP = B_PER_W // GRP
GHW = GRP * HW
IDXW = 5120
ACCW = 640
NCHUNK = ACCW // L


def _sc_body(i0, i1, i2, i3, i4, i5, tab_hbm, out_hbm,
             tab_v, t1_v, t2_v, idx_v, acc_v):
    wid = lax.axis_index("s") * NC + lax.axis_index("c")
    idx_refs = (i0, i1, i2, i3, i4, i5)

    pltpu.sync_copy(tab_hbm, tab_v)
    iota = lax.iota(jnp.int32, L)

    def build_t1(e, _):
        e_vec = jnp.full((L,), 0, jnp.int32) + e
        for jc in range(4):
            j = jnp.minimum(iota + (jc * L), 62)
            c0 = j // 21
            r = j - c0 * 21
            c1 = r // 3
            c2 = r - c1 * 3
            v = (plsc.load_gather(tab_v, [c0, e_vec])
                 + plsc.load_gather(tab_v, [c1 + 3, e_vec])
                 + plsc.load_gather(tab_v, [c2 + 10, e_vec]))
            t1_v[e, pl.ds(jc * L, L)] = v
        return _

    def build_t2(e, _):
        e_vec = jnp.full((L,), 0, jnp.int32) + e
        for jc in range(7):
            j = jnp.minimum(iota + (jc * L), 107)
            c3 = j // 36
            r = j - c3 * 36
            c4 = r // 6
            c5 = r - c4 * 6
            v = (plsc.load_gather(tab_v, [c3 + 13, e_vec])
                 + plsc.load_gather(tab_v, [c4 + 16, e_vec])
                 + plsc.load_gather(tab_v, [c5 + 22, e_vec]))
            t2_v[e, pl.ds(jc * L, L)] = v
        return _

    lax.fori_loop(0, EMB, build_t1, 0)
    lax.fori_loop(0, EMB, build_t2, 0)

    zero = jnp.full((L,), 0, jnp.int32)
    for p in range(6):
        idx_v[p, pl.ds(GHW, L)] = zero

    def group_body(g, _):
        base = wid * (B_PER_W * HW) + g * GHW
        for p in range(6):
            pltpu.sync_copy(idx_refs[p].at[pl.ds(base, GHW)],
                            idx_v.at[p, pl.ds(0, GHW)])

        def row_body(sub, _):
            off = sub * HW

            def chunk_body(c, _):
                s = off + c * L
                v0 = idx_v[0, pl.ds(s, L)]
                v1 = idx_v[1, pl.ds(s, L)]
                v2 = idx_v[2, pl.ds(s, L)]
                v3 = idx_v[3, pl.ds(s, L)]
                v4 = idx_v[4, pl.ds(s, L)]
                v5 = idx_v[5, pl.ds(s, L)]
                j1 = (v0 * 7 + v1) * 3 + v2
                j2 = (v3 * 6 + v4) * 6 + v5
                d = c * L
                for e in range(EMB):
                    val = (plsc.load_gather(t1_v.at[e], [j1])
                           + plsc.load_gather(t2_v.at[e], [j2]))
                    acc_v[e, pl.ds(d, L)] = val
                return _

            lax.fori_loop(0, NCHUNK, chunk_body, 0)
            b = wid * B_PER_W + g * GRP + sub
            pltpu.sync_copy(acc_v.at[:, pl.ds(0, HW)], out_hbm.at[b])
            return _

        lax.fori_loop(0, GRP, row_body, 0)
        return _

    lax.fori_loop(0, NGRP, group_body, 0)


@jax.jit
def _run(i0, i1, i2, i3, i4, i5, tab):
    mesh = plsc.VectorSubcoreMesh(core_axis_name="c", subcore_axis_name="s")
    f = pl.kernel(
        _sc_body,
        out_type=jax.ShapeDtypeStruct((B, EMB, HW), jnp.float32),
        mesh=mesh,
        scratch_types=[
            pltpu.VMEM((28, EMB), jnp.float32),
            pltpu.VMEM((EMB, 64), jnp.float32),
            pltpu.VMEM((EMB, 112), jnp.float32),
            pltpu.VMEM((6, IDXW), jnp.int32),
            pltpu.VMEM((EMB, ACCW), jnp.float32),
        ],
    )
    return f(i0, i1, i2, i3, i4, i5, tab)


def kernel(card_counts, card_colors, card_shapes, card_selections,
           leader_rotations, follower_rotations, embedding_table):
    flat = lambda x: x.reshape(-1).astype(jnp.int32)
    out = _run(flat(card_counts), flat(card_colors), flat(card_shapes),
               flat(card_selections), flat(leader_rotations),
               flat(follower_rotations), embedding_table.astype(jnp.float32))
    return out.reshape(B, EMB, H, W)


# speedup vs baseline: 39.6386x; 39.6386x over previous
import functools
import jax
import jax.numpy as jnp
from jax import lax
from jax.experimental import pallas as pl
from jax.experimental.pallas import tpu as pltpu
from jax.experimental.pallas import tpu_sc as plsc

B, H, W, EMB = 1024, 25, 25, 32
HW = H * W
NC, NS, L = 2, 16, 16
NW = NC * NS
B_PER_W = B // NW
GRP = 8
NGRP = B_PER_W // GRP
GHW = GRP * HW
IDXW = 5120
NFULL = HW // L
TAILS = HW - L


def _sc_body(i0, i1, i2, i3, i4, i5, tab_hbm, out_hbm,
             tab_v, t1_v, t2_v, idx_v, acc_v):
    wid = lax.axis_index("s") * NC + lax.axis_index("c")
    idx_refs = (i0, i1, i2, i3, i4, i5)

    pltpu.sync_copy(tab_hbm, tab_v)
    iota = lax.iota(jnp.int32, L)

    def build_t1(e, _):
        e_vec = jnp.full((L,), 0, jnp.int32) + e
        for jc in range(4):
            j = jnp.minimum(iota + (jc * L), 62)
            c0 = j // 21
            r = j - c0 * 21
            c1 = r // 3
            c2 = r - c1 * 3
            v = (plsc.load_gather(tab_v, [c0, e_vec])
                 + plsc.load_gather(tab_v, [c1 + 3, e_vec])
                 + plsc.load_gather(tab_v, [c2 + 10, e_vec]))
            t1_v[e, pl.ds(jc * L, L)] = v
        return _

    def build_t2(e, _):
        e_vec = jnp.full((L,), 0, jnp.int32) + e
        for jc in range(7):
            j = jnp.minimum(iota + (jc * L), 107)
            c3 = j // 36
            r = j - c3 * 36
            c4 = r // 6
            c5 = r - c4 * 6
            v = (plsc.load_gather(tab_v, [c3 + 13, e_vec])
                 + plsc.load_gather(tab_v, [c4 + 16, e_vec])
                 + plsc.load_gather(tab_v, [c5 + 22, e_vec]))
            t2_v[e, pl.ds(jc * L, L)] = v
        return _

    lax.fori_loop(0, EMB, build_t1, 0)
    lax.fori_loop(0, EMB, build_t2, 0)

    def compute_chunk(off, d):
        s = off + d
        v0 = idx_v[0, pl.ds(s, L)]
        v1 = idx_v[1, pl.ds(s, L)]
        v2 = idx_v[2, pl.ds(s, L)]
        v3 = idx_v[3, pl.ds(s, L)]
        v4 = idx_v[4, pl.ds(s, L)]
        v5 = idx_v[5, pl.ds(s, L)]
        j1 = (v0 * 7 + v1) * 3 + v2
        j2 = (v3 * 6 + v4) * 6 + v5
        for e in range(EMB):
            val = (plsc.load_gather(t1_v.at[e], [j1])
                   + plsc.load_gather(t2_v.at[e], [j2]))
            acc_v[e, pl.ds(d, L)] = val

    def group_body(g, _):
        base = wid * (B_PER_W * HW) + g * GHW
        for p in range(6):
            pltpu.sync_copy(idx_refs[p].at[pl.ds(base, GHW)],
                            idx_v.at[p, pl.ds(0, GHW)])

        def row_body(sub, _):
            off = sub * HW

            def chunk_body(c, _):
                compute_chunk(off, c * L)
                return _

            lax.fori_loop(0, NFULL, chunk_body, 0)
            compute_chunk(off, TAILS)
            b = wid * B_PER_W + g * GRP + sub
            pltpu.sync_copy(acc_v, out_hbm.at[b])
            return _

        lax.fori_loop(0, GRP, row_body, 0)
        return _

    lax.fori_loop(0, NGRP, group_body, 0)


@jax.jit
def _run(i0, i1, i2, i3, i4, i5, tab):
    mesh = plsc.VectorSubcoreMesh(core_axis_name="c", subcore_axis_name="s")
    f = pl.kernel(
        _sc_body,
        out_type=jax.ShapeDtypeStruct((B, EMB, HW), jnp.float32),
        mesh=mesh,
        compiler_params=pltpu.CompilerParams(
            use_tc_tiling_on_sc=False, needs_layout_passes=False),
        scratch_types=[
            pltpu.VMEM((28, EMB), jnp.float32),
            pltpu.VMEM((EMB, 64), jnp.float32),
            pltpu.VMEM((EMB, 112), jnp.float32),
            pltpu.VMEM((6, IDXW), jnp.int32),
            pltpu.VMEM((EMB, HW), jnp.float32),
        ],
    )
    return f(i0, i1, i2, i3, i4, i5, tab)


def kernel(card_counts, card_colors, card_shapes, card_selections,
           leader_rotations, follower_rotations, embedding_table):
    flat = lambda x: x.reshape(-1).astype(jnp.int32)
    out = _run(flat(card_counts), flat(card_colors), flat(card_shapes),
               flat(card_selections), flat(leader_rotations),
               flat(follower_rotations), embedding_table.astype(jnp.float32))
    return out.reshape(B, EMB, H, W)
